# baseline (speedup 1.0000x reference)
import functools

import jax
import jax.numpy as jnp
from jax import lax
from jax.experimental import pallas as pl
from jax.experimental.pallas import tpu as pltpu
from jax.experimental.pallas import tpu_sc as plsc

B = 16384
D = 32
NNEG = 20
MARGIN = 1.0
L = 16
NC, NS = 2, 16
NW = NC * NS
PER_W = B // NW
N_GROUPS = PER_W // L
BLK_E = 32
N_BLK = PER_W // BLK_E
GRP_PER_BLK = BLK_E // L
BLK_ROWS = BLK_E * NNEG
IDX_CHUNK = 128
N_NCHUNK = BLK_ROWS // IDX_CHUNK
JC = 5
NJC = NNEG // JC


N_ROWS = 1000000
TILE_R = 128
N_TILES = N_ROWS // TILE_R
PART_ROWS = N_ROWS - N_TILES * TILE_R
TPC = 4
CHUNK_R = TPC * TILE_R
CHUNK_W = CHUNK_R * D
N_CHUNK = N_TILES // TPC
CPW = N_CHUNK // NW
REM_C = N_CHUNK - CPW * NW


def _tr_tile(src_v, dst1, n_r16):
    iota = lax.iota(jnp.int32, L)
    iota32 = iota * 32
    dm = [(iota + s) & 15 for s in range(16)]
    fw = [iota32 + dm[s] for s in range(16)]

    def rb_body(rb16, _):
        r_vec = iota + rb16 * 16
        woff = rb16 * 512
        for s4 in range(2):
            vals = []
            for k in range(16):
                s, db = s4 * 8 + k // 2, k % 2
                vals.append(plsc.load_gather(
                    src_v, [dm[s] + db * 16 if db else dm[s], r_vec]))
            for k in range(16):
                s, db = s4 * 8 + k // 2, k % 2
                plsc.store_scatter(dst1, [fw[s] + (woff + db * 16)], vals[k])
        return 0

    lax.fori_loop(0, n_r16, rb_body, 0)


def _tr_body(ut, it, tu_out, ti_out,
             s0, s1, d0, d1, sp, semi0, semi1, semo0, semo1):
    cid = lax.axis_index("c")
    sid = lax.axis_index("s")
    wid = sid * NC + cid
    base_c = wid * CPW

    for tab, out in ((ut, tu_out), (it, ti_out)):
        def fire_in(c, buf, sem):
            pltpu.async_copy(tab.at[:, pl.ds(c * CHUNK_R, CHUNK_R)], buf, sem)

        def drain(buf, sem):
            pltpu.make_async_copy(tab.at[:, pl.ds(0, CHUNK_R)], buf, sem).wait()

        def drain_out(buf, sem):
            pltpu.make_async_copy(buf, out.at[pl.ds(0, CHUNK_W)], sem).wait()

        fire_in(base_c + 0, s0, semi0)
        fire_in(base_c + 1, s1, semi1)

        def pair_body(i2, _):
            for (off, s_v, d_v, si, so) in ((0, s0, d0, semi0, semo0),
                                            (1, s1, d1, semi1, semo1)):
                c = base_c + 2 * i2 + off
                drain(s_v, si)

                @pl.when(i2 > 0)
                def _():
                    drain_out(d_v, so)

                _tr_tile(s_v, d_v, CHUNK_R // 16)
                pltpu.async_copy(d_v, out.at[pl.ds(c * CHUNK_W, CHUNK_W)], so)

                @pl.when(i2 < CPW // 2 - 1)
                def _():
                    fire_in(c + 2, s_v, si)

            return 0

        lax.fori_loop(0, CPW // 2, pair_body, 0)
        drain_out(d0, semo0)
        drain_out(d1, semo1)

        c_last = base_c + CPW - 1
        pltpu.sync_copy(tab.at[:, pl.ds(c_last * CHUNK_R, CHUNK_R)], s0)
        _tr_tile(s0, d0, CHUNK_R // 16)
        pltpu.sync_copy(d0, out.at[pl.ds(c_last * CHUNK_W, CHUNK_W)])

        @pl.when(wid < REM_C)
        def _():
            c = NW * CPW + wid
            pltpu.sync_copy(tab.at[:, pl.ds(c * CHUNK_R, CHUNK_R)], s0)
            _tr_tile(s0, d0, CHUNK_R // 16)
            pltpu.sync_copy(d0, out.at[pl.ds(c * CHUNK_W, CHUNK_W)])

        @pl.when(wid == NW - 1)
        def _():
            pltpu.sync_copy(tab.at[:, pl.ds(N_TILES * TILE_R, PART_ROWS)], sp)
            _tr_tile(sp, d0, PART_ROWS // 16)
            pltpu.sync_copy(d0.at[pl.ds(0, PART_ROWS * D)],
                            out.at[pl.ds(N_TILES * TILE_R * D, PART_ROWS * D)])


def _sc_body(aid_hbm, pid_hbm, nid_hbm, utab, itab, out_hbm,
             aid_v, pid_v, nid_v, a_rows, p_rows, n_buf0, n_buf1,
             part_v, sem_ap, sem_n0, sem_n1):
    cid = lax.axis_index("c")
    sid = lax.axis_index("s")
    wid = sid * NC + cid

    pltpu.sync_copy(aid_hbm.at[pl.ds(wid * (PER_W // 128), PER_W // 128), :], aid_v)
    pltpu.sync_copy(pid_hbm.at[pl.ds(wid * (PER_W // 128), PER_W // 128), :], pid_v)
    pltpu.sync_copy(nid_hbm.at[pl.ds(wid * (PER_W * NNEG // 128), PER_W * NNEG // 128), :], nid_v)

    for k in range(PER_W // IDX_CHUNK):
        pltpu.async_copy(utab.at[aid_v.at[k]],
                         a_rows.at[pl.ds(k * IDX_CHUNK, IDX_CHUNK), :], sem_ap)
        pltpu.async_copy(itab.at[pid_v.at[k]],
                         p_rows.at[pl.ds(k * IDX_CHUNK, IDX_CHUNK), :], sem_ap)

    def fire_nblock(blk, buf, sem):
        for c in range(N_NCHUNK):
            pltpu.async_copy(itab.at[nid_v.at[blk * N_NCHUNK + c]],
                             buf.at[pl.ds(c * IDX_CHUNK, IDX_CHUNK), :], sem)

    def drain(dst, sem):
        pltpu.make_async_copy(itab.at[pl.ds(0, dst.shape[0]), :], dst, sem).wait()

    fire_nblock(0, n_buf0, sem_n0)
    fire_nblock(1, n_buf1, sem_n1)
    drain(a_rows, sem_ap)
    drain(p_rows, sem_ap)

    iota = lax.iota(jnp.int32, L)
    e20 = iota * NNEG
    zero = jnp.zeros((L,), jnp.float32)

    def group_compute(g, gl, n_buf, acc):
        a_idx = iota + g * L

        def pd_body(d, carry):
            aa, pp, dap = carry
            dv = (iota + d) & (D - 1)
            a_d = plsc.load_gather(a_rows, [a_idx, dv])
            p_d = plsc.load_gather(p_rows, [a_idx, dv])
            return (aa + a_d * a_d, pp + p_d * p_d, dap + a_d * p_d)

        aa, pp, dap = lax.fori_loop(0, D, pd_body, (zero, zero, zero))
        pos_d = jnp.maximum(aa + pp - 2.0 * dap, 0.0)

        mn = jnp.full((L,), 1e30, jnp.float32)
        mda = jnp.full((L,), -1e30, jnp.float32)
        mdp = jnp.full((L,), -1e30, jnp.float32)
        for jc in range(NJC):
            idxs = [e20 + (gl * BLK_ROWS // GRP_PER_BLK + jc * JC + jj)
                    for jj in range(JC)]

            def nd_body(d, carry, idxs=idxs):
                dv = (iota + d) & (D - 1)
                a_d = plsc.load_gather(a_rows, [a_idx, dv])
                p_d = plsc.load_gather(p_rows, [a_idx, dv])
                nn, da, dp = list(carry[:JC]), list(carry[JC:2 * JC]), list(carry[2 * JC:])
                for jj in range(JC):
                    n = plsc.load_gather(n_buf, [idxs[jj], dv])
                    nn[jj] = nn[jj] + n * n
                    da[jj] = da[jj] + a_d * n
                    dp[jj] = dp[jj] + p_d * n
                return tuple(nn + da + dp)

            res = lax.fori_loop(0, D, nd_body, (zero,) * (3 * JC))
            for jj in range(JC):
                nn_j, da_j, dp_j = res[jj], res[JC + jj], res[2 * JC + jj]
                nd = aa + nn_j - 2.0 * da_j
                mn = jnp.minimum(mn, nd)
                mda = jnp.maximum(mda, da_j)
                mdp = jnp.maximum(mdp, dp_j)

        pair, m1a, m2a, m1p, m2p = acc
        pair = pair + jnp.maximum(pos_d - jnp.maximum(mn, 0.0) + MARGIN, 0.0)
        return (pair, m1a + mda, m2a + mda * mda, m1p + mdp, m2p + mdp * mdp)

    def blk_body(bb, acc):
        drain(n_buf0, sem_n0)
        for gl in range(GRP_PER_BLK):
            acc = group_compute((2 * bb) * GRP_PER_BLK + gl, gl, n_buf0, acc)

        @pl.when(bb < N_BLK // 2 - 1)
        def _():
            fire_nblock(2 * bb + 2, n_buf0, sem_n0)

        drain(n_buf1, sem_n1)
        for gl in range(GRP_PER_BLK):
            acc = group_compute((2 * bb + 1) * GRP_PER_BLK + gl, gl, n_buf1, acc)

        @pl.when(bb < N_BLK // 2 - 1)
        def _():
            fire_nblock(2 * bb + 3, n_buf1, sem_n1)

        return acc

    acc = lax.fori_loop(0, N_BLK // 2, blk_body, (zero,) * 5)

    pair, m1a, m2a, m1p, m2p = acc
    part_v[0, :] = pair
    part_v[1, :] = m1a
    part_v[2, :] = m2a
    part_v[3, :] = m1p
    part_v[4, :] = m2p
    pltpu.sync_copy(part_v, out_hbm.at[wid])


_tr_call = pl.kernel(
    _tr_body,
    out_type=(jax.ShapeDtypeStruct((N_ROWS * D,), jnp.float32),
              jax.ShapeDtypeStruct((N_ROWS * D,), jnp.float32)),
    mesh=plsc.VectorSubcoreMesh(core_axis_name="c", subcore_axis_name="s",
                                num_cores=NC, num_subcores=NS),
    scratch_types=[
        pltpu.VMEM((D, CHUNK_R), jnp.float32),
        pltpu.VMEM((D, CHUNK_R), jnp.float32),
        pltpu.VMEM((CHUNK_W,), jnp.float32),
        pltpu.VMEM((CHUNK_W,), jnp.float32),
        pltpu.VMEM((D, PART_ROWS), jnp.float32),
        pltpu.SemaphoreType.DMA,
        pltpu.SemaphoreType.DMA,
        pltpu.SemaphoreType.DMA,
        pltpu.SemaphoreType.DMA,
    ],
    compiler_params=pltpu.CompilerParams(needs_layout_passes=False,
                                         use_tc_tiling_on_sc=True),
)

_sc_call = pl.kernel(
    _sc_body,
    out_type=jax.ShapeDtypeStruct((NW, 5, L), jnp.float32),
    mesh=plsc.VectorSubcoreMesh(core_axis_name="c", subcore_axis_name="s",
                                num_cores=NC, num_subcores=NS),
    compiler_params=pltpu.CompilerParams(needs_layout_passes=False,
                                         use_tc_tiling_on_sc=False),
    scratch_types=[
        pltpu.VMEM((PER_W // 128, 128), jnp.int32),
        pltpu.VMEM((PER_W // 128, 128), jnp.int32),
        pltpu.VMEM((PER_W * NNEG // 128, 128), jnp.int32),
        pltpu.VMEM((PER_W, D), jnp.float32),
        pltpu.VMEM((PER_W, D), jnp.float32),
        pltpu.VMEM((BLK_ROWS, D), jnp.float32),
        pltpu.VMEM((BLK_ROWS, D), jnp.float32),
        pltpu.VMEM((5, L), jnp.float32),
        pltpu.SemaphoreType.DMA,
        pltpu.SemaphoreType.DMA,
        pltpu.SemaphoreType.DMA,
    ],
)


def kernel(anchor_ids, pos_ids, neg_ids, user_embeddings, item_embeddings):
    aid2 = anchor_ids.reshape(B // 128, 128)
    pid2 = pos_ids.reshape(B // 128, 128)
    nid2 = neg_ids.reshape(B * NNEG // 128, 128)
    tu, ti = _tr_call(user_embeddings.T, item_embeddings.T)
    parts = _sc_call(aid2, pid2, nid2,
                     tu.reshape(N_ROWS, D), ti.reshape(N_ROWS, D))
    pair = jnp.sum(parts[:, 0, :])
    m1a = jnp.sum(parts[:, 1, :])
    m2a = jnp.sum(parts[:, 2, :])
    m1p = jnp.sum(parts[:, 3, :])
    m2p = jnp.sum(parts[:, 4, :])
    n_over_d = jnp.float32(B) / jnp.float32(D)
    sa = jnp.maximum(m1a * m1a + jnp.maximum(0.0, m2a - n_over_d), 0.0)
    sp = jnp.maximum(m1p * m1p + jnp.maximum(0.0, m2p - n_over_d), 0.0)
    return pair + sa + sp


# speedup vs baseline: 1.0061x; 1.0061x over previous
import functools

import jax
import jax.numpy as jnp
from jax import lax
from jax.experimental import pallas as pl
from jax.experimental.pallas import tpu as pltpu
from jax.experimental.pallas import tpu_sc as plsc

B = 16384
D = 32
NNEG = 20
MARGIN = 1.0
L = 16
NC, NS = 2, 16
NW = NC * NS
PER_W = B // NW
N_GROUPS = PER_W // L
BLK_E = 32
N_BLK = PER_W // BLK_E
GRP_PER_BLK = BLK_E // L
BLK_ROWS = BLK_E * NNEG
IDX_CHUNK = 128
N_NCHUNK = BLK_ROWS // IDX_CHUNK
JC = 5
NJC = NNEG // JC


N_ROWS = 1000000
TILE_R = 128
N_TILES = N_ROWS // TILE_R
PART_ROWS = N_ROWS - N_TILES * TILE_R
TPC = 4
CHUNK_R = TPC * TILE_R
CHUNK_W = CHUNK_R * D
N_CHUNK = N_TILES // TPC
CPW = N_CHUNK // NW
REM_C = N_CHUNK - CPW * NW


def _tr_tile(src_v, dst1, n_r16):
    iota = lax.iota(jnp.int32, L)
    iota32 = iota * 32
    dm = [(iota + s) & 15 for s in range(16)]
    fw = [iota32 + dm[s] for s in range(16)]

    def rb_body(rb16, _):
        r_vec = iota + rb16 * 16
        woff = rb16 * 512
        for s4 in range(4):
            vals = []
            for k in range(8):
                s, db = s4 * 4 + k // 2, k % 2
                vals.append(plsc.load_gather(
                    src_v, [dm[s] + db * 16 if db else dm[s], r_vec]))
            for k in range(8):
                s, db = s4 * 4 + k // 2, k % 2
                plsc.store_scatter(dst1, [fw[s] + (woff + db * 16)], vals[k])
        return 0

    lax.fori_loop(0, n_r16, rb_body, 0)


def _tr_body(ut, it, tu_out, ti_out,
             s0, s1, d0, d1, sp, semi0, semi1, semo0, semo1):
    cid = lax.axis_index("c")
    sid = lax.axis_index("s")
    wid = sid * NC + cid
    base_c = wid * CPW

    for tab, out in ((ut, tu_out), (it, ti_out)):
        def fire_in(c, buf, sem):
            pltpu.async_copy(tab.at[:, pl.ds(c * CHUNK_R, CHUNK_R)], buf, sem)

        def drain(buf, sem):
            pltpu.make_async_copy(tab.at[:, pl.ds(0, CHUNK_R)], buf, sem).wait()

        def drain_out(buf, sem):
            pltpu.make_async_copy(buf, out.at[pl.ds(0, CHUNK_W)], sem).wait()

        fire_in(base_c + 0, s0, semi0)
        fire_in(base_c + 1, s1, semi1)

        def pair_body(i2, _):
            for (off, s_v, d_v, si, so) in ((0, s0, d0, semi0, semo0),
                                            (1, s1, d1, semi1, semo1)):
                c = base_c + 2 * i2 + off
                drain(s_v, si)

                @pl.when(i2 > 0)
                def _():
                    drain_out(d_v, so)

                _tr_tile(s_v, d_v, CHUNK_R // 16)
                pltpu.async_copy(d_v, out.at[pl.ds(c * CHUNK_W, CHUNK_W)], so)

                @pl.when(i2 < CPW // 2 - 1)
                def _():
                    fire_in(c + 2, s_v, si)

            return 0

        lax.fori_loop(0, CPW // 2, pair_body, 0)
        drain_out(d0, semo0)
        drain_out(d1, semo1)

        c_last = base_c + CPW - 1
        pltpu.sync_copy(tab.at[:, pl.ds(c_last * CHUNK_R, CHUNK_R)], s0)
        _tr_tile(s0, d0, CHUNK_R // 16)
        pltpu.sync_copy(d0, out.at[pl.ds(c_last * CHUNK_W, CHUNK_W)])

        @pl.when(wid < REM_C)
        def _():
            c = NW * CPW + wid
            pltpu.sync_copy(tab.at[:, pl.ds(c * CHUNK_R, CHUNK_R)], s0)
            _tr_tile(s0, d0, CHUNK_R // 16)
            pltpu.sync_copy(d0, out.at[pl.ds(c * CHUNK_W, CHUNK_W)])

        @pl.when(wid == NW - 1)
        def _():
            pltpu.sync_copy(tab.at[:, pl.ds(N_TILES * TILE_R, PART_ROWS)], sp)
            _tr_tile(sp, d0, PART_ROWS // 16)
            pltpu.sync_copy(d0.at[pl.ds(0, PART_ROWS * D)],
                            out.at[pl.ds(N_TILES * TILE_R * D, PART_ROWS * D)])


def _sc_body(aid_hbm, pid_hbm, nid_hbm, utab, itab, out_hbm,
             aid_v, pid_v, nid_v, a_rows, p_rows, n_buf0, n_buf1,
             part_v, sem_ap, sem_n0, sem_n1):
    cid = lax.axis_index("c")
    sid = lax.axis_index("s")
    wid = sid * NC + cid

    pltpu.sync_copy(aid_hbm.at[pl.ds(wid * (PER_W // 128), PER_W // 128), :], aid_v)
    pltpu.sync_copy(pid_hbm.at[pl.ds(wid * (PER_W // 128), PER_W // 128), :], pid_v)
    pltpu.sync_copy(nid_hbm.at[pl.ds(wid * (PER_W * NNEG // 128), PER_W * NNEG // 128), :], nid_v)

    for k in range(PER_W // IDX_CHUNK):
        pltpu.async_copy(utab.at[aid_v.at[k]],
                         a_rows.at[pl.ds(k * IDX_CHUNK, IDX_CHUNK), :], sem_ap)
        pltpu.async_copy(itab.at[pid_v.at[k]],
                         p_rows.at[pl.ds(k * IDX_CHUNK, IDX_CHUNK), :], sem_ap)

    def fire_nblock(blk, buf, sem):
        for c in range(N_NCHUNK):
            pltpu.async_copy(itab.at[nid_v.at[blk * N_NCHUNK + c]],
                             buf.at[pl.ds(c * IDX_CHUNK, IDX_CHUNK), :], sem)

    def drain(dst, sem):
        pltpu.make_async_copy(itab.at[pl.ds(0, dst.shape[0]), :], dst, sem).wait()

    fire_nblock(0, n_buf0, sem_n0)
    fire_nblock(1, n_buf1, sem_n1)
    drain(a_rows, sem_ap)
    drain(p_rows, sem_ap)

    iota = lax.iota(jnp.int32, L)
    e20 = iota * NNEG
    zero = jnp.zeros((L,), jnp.float32)

    def group_compute(g, gl, n_buf, acc):
        a_idx = iota + g * L

        def pd_body(d, carry):
            aa, pp, dap = carry
            dv = (iota + d) & (D - 1)
            a_d = plsc.load_gather(a_rows, [a_idx, dv])
            p_d = plsc.load_gather(p_rows, [a_idx, dv])
            return (aa + a_d * a_d, pp + p_d * p_d, dap + a_d * p_d)

        aa, pp, dap = lax.fori_loop(0, D, pd_body, (zero, zero, zero))
        pos_d = jnp.maximum(aa + pp - 2.0 * dap, 0.0)

        mn = jnp.full((L,), 1e30, jnp.float32)
        mda = jnp.full((L,), -1e30, jnp.float32)
        mdp = jnp.full((L,), -1e30, jnp.float32)
        for jc in range(NJC):
            idxs = [e20 + (gl * BLK_ROWS // GRP_PER_BLK + jc * JC + jj)
                    for jj in range(JC)]

            def nd_body(d, carry, idxs=idxs):
                dv = (iota + d) & (D - 1)
                a_d = plsc.load_gather(a_rows, [a_idx, dv])
                p_d = plsc.load_gather(p_rows, [a_idx, dv])
                nn, da, dp = list(carry[:JC]), list(carry[JC:2 * JC]), list(carry[2 * JC:])
                for jj in range(JC):
                    n = plsc.load_gather(n_buf, [idxs[jj], dv])
                    nn[jj] = nn[jj] + n * n
                    da[jj] = da[jj] + a_d * n
                    dp[jj] = dp[jj] + p_d * n
                return tuple(nn + da + dp)

            res = lax.fori_loop(0, D, nd_body, (zero,) * (3 * JC))
            for jj in range(JC):
                nn_j, da_j, dp_j = res[jj], res[JC + jj], res[2 * JC + jj]
                nd = aa + nn_j - 2.0 * da_j
                mn = jnp.minimum(mn, nd)
                mda = jnp.maximum(mda, da_j)
                mdp = jnp.maximum(mdp, dp_j)

        pair, m1a, m2a, m1p, m2p = acc
        pair = pair + jnp.maximum(pos_d - jnp.maximum(mn, 0.0) + MARGIN, 0.0)
        return (pair, m1a + mda, m2a + mda * mda, m1p + mdp, m2p + mdp * mdp)

    def blk_body(bb, acc):
        drain(n_buf0, sem_n0)
        for gl in range(GRP_PER_BLK):
            acc = group_compute((2 * bb) * GRP_PER_BLK + gl, gl, n_buf0, acc)

        @pl.when(bb < N_BLK // 2 - 1)
        def _():
            fire_nblock(2 * bb + 2, n_buf0, sem_n0)

        drain(n_buf1, sem_n1)
        for gl in range(GRP_PER_BLK):
            acc = group_compute((2 * bb + 1) * GRP_PER_BLK + gl, gl, n_buf1, acc)

        @pl.when(bb < N_BLK // 2 - 1)
        def _():
            fire_nblock(2 * bb + 3, n_buf1, sem_n1)

        return acc

    acc = lax.fori_loop(0, N_BLK // 2, blk_body, (zero,) * 5)

    pair, m1a, m2a, m1p, m2p = acc
    part_v[0, :] = pair
    part_v[1, :] = m1a
    part_v[2, :] = m2a
    part_v[3, :] = m1p
    part_v[4, :] = m2p
    pltpu.sync_copy(part_v, out_hbm.at[wid])


_tr_call = pl.kernel(
    _tr_body,
    out_type=(jax.ShapeDtypeStruct((N_ROWS * D,), jnp.float32),
              jax.ShapeDtypeStruct((N_ROWS * D,), jnp.float32)),
    mesh=plsc.VectorSubcoreMesh(core_axis_name="c", subcore_axis_name="s",
                                num_cores=NC, num_subcores=NS),
    scratch_types=[
        pltpu.VMEM((D, CHUNK_R), jnp.float32),
        pltpu.VMEM((D, CHUNK_R), jnp.float32),
        pltpu.VMEM((CHUNK_W,), jnp.float32),
        pltpu.VMEM((CHUNK_W,), jnp.float32),
        pltpu.VMEM((D, PART_ROWS), jnp.float32),
        pltpu.SemaphoreType.DMA,
        pltpu.SemaphoreType.DMA,
        pltpu.SemaphoreType.DMA,
        pltpu.SemaphoreType.DMA,
    ],
    compiler_params=pltpu.CompilerParams(needs_layout_passes=False,
                                         use_tc_tiling_on_sc=True),
)

_sc_call = pl.kernel(
    _sc_body,
    out_type=jax.ShapeDtypeStruct((NW, 5, L), jnp.float32),
    mesh=plsc.VectorSubcoreMesh(core_axis_name="c", subcore_axis_name="s",
                                num_cores=NC, num_subcores=NS),
    compiler_params=pltpu.CompilerParams(needs_layout_passes=False,
                                         use_tc_tiling_on_sc=False),
    scratch_types=[
        pltpu.VMEM((PER_W // 128, 128), jnp.int32),
        pltpu.VMEM((PER_W // 128, 128), jnp.int32),
        pltpu.VMEM((PER_W * NNEG // 128, 128), jnp.int32),
        pltpu.VMEM((PER_W, D), jnp.float32),
        pltpu.VMEM((PER_W, D), jnp.float32),
        pltpu.VMEM((BLK_ROWS, D), jnp.float32),
        pltpu.VMEM((BLK_ROWS, D), jnp.float32),
        pltpu.VMEM((5, L), jnp.float32),
        pltpu.SemaphoreType.DMA,
        pltpu.SemaphoreType.DMA,
        pltpu.SemaphoreType.DMA,
    ],
)


def kernel(anchor_ids, pos_ids, neg_ids, user_embeddings, item_embeddings):
    aid2 = anchor_ids.reshape(B // 128, 128)
    pid2 = pos_ids.reshape(B // 128, 128)
    nid2 = neg_ids.reshape(B * NNEG // 128, 128)
    tu, ti = _tr_call(user_embeddings.T, item_embeddings.T)
    parts = _sc_call(aid2, pid2, nid2,
                     tu.reshape(N_ROWS, D), ti.reshape(N_ROWS, D))
    pair = jnp.sum(parts[:, 0, :])
    m1a = jnp.sum(parts[:, 1, :])
    m2a = jnp.sum(parts[:, 2, :])
    m1p = jnp.sum(parts[:, 3, :])
    m2p = jnp.sum(parts[:, 4, :])
    n_over_d = jnp.float32(B) / jnp.float32(D)
    sa = jnp.maximum(m1a * m1a + jnp.maximum(0.0, m2a - n_over_d), 0.0)
    sp = jnp.maximum(m1p * m1p + jnp.maximum(0.0, m2p - n_over_d), 0.0)
    return pair + sa + sp
